# baseline (speedup 1.0000x reference)
import jax
import jax.numpy as jnp
from jax.experimental import pallas as pl
from jax.experimental.pallas import tpu as pltpu


def _mmoe_body(x_ref, we_ref, be_ref, wg_ref, wt_ref, bt_ref, out_ref, gates_ref):
    E = gates_ref.shape[0]
    T = gates_ref.shape[2]
    e = pl.program_id(0)
    j = pl.program_id(1)
    nj = pl.num_programs(1)

    @pl.when(jnp.logical_and(e == 0, j == 0))
    def _init():
        gm = jnp.dot(x_ref[...].astype(jnp.float32), wg_ref[...],
                     preferred_element_type=jnp.float32)
        for t in range(T):
            lg = gm[:, t * E:(t + 1) * E]
            m = jnp.max(lg, axis=1, keepdims=True)
            ex = jnp.exp(lg - m)
            sm = ex / jnp.sum(ex, axis=1, keepdims=True)
            for e2 in range(E):
                gates_ref[e2, :, t:t + 1] = sm[:, e2:e2 + 1]
        out_ref[...] = jnp.zeros_like(out_ref)

    h = jnp.maximum(
        jnp.dot(x_ref[...], we_ref[0], preferred_element_type=jnp.float32)
        + be_ref[0], 0.0)
    p = jnp.dot(h, wt_ref[...])
    out_ref[...] += gates_ref[e] * p

    @pl.when(jnp.logical_and(e == E - 1, j == nj - 1))
    def _fini():
        out_ref[...] += bt_ref[...]


def kernel(x, We, be, Wg, Wt, bt):
    N, D = x.shape
    E, _, F = We.shape
    T = Wg.shape[0]
    bf = min(512, F)
    J = F // bf

    wg_flat = jnp.transpose(Wg, (1, 0, 2)).reshape(D, T * E)
    wt_flat = jnp.transpose(Wt[:, :, 0], (1, 0))
    be3 = be.reshape(E, 1, F)
    bt_row = bt.reshape(1, T)
    xb = x.astype(jnp.bfloat16)
    web = We.astype(jnp.bfloat16)

    out = pl.pallas_call(
        _mmoe_body,
        grid=(E, J),
        in_specs=[
            pl.BlockSpec((N, D), lambda e, j: (0, 0)),
            pl.BlockSpec((1, D, bf), lambda e, j: (e, 0, j)),
            pl.BlockSpec((1, 1, bf), lambda e, j: (e, 0, j)),
            pl.BlockSpec((D, T * E), lambda e, j: (0, 0)),
            pl.BlockSpec((bf, T), lambda e, j: (j, 0)),
            pl.BlockSpec((1, T), lambda e, j: (0, 0)),
        ],
        out_specs=pl.BlockSpec((N, T), lambda e, j: (0, 0)),
        out_shape=jax.ShapeDtypeStruct((N, T), jnp.float32),
        scratch_shapes=[pltpu.VMEM((E, N, T), jnp.float32)],
        compiler_params=pltpu.CompilerParams(
            dimension_semantics=("arbitrary", "arbitrary"),
        ),
    )(xb, web, be3, wg_flat, wt_flat, bt_row)

    return jnp.transpose(out)[:, :, None]


# speedup vs baseline: 1.3791x; 1.3791x over previous
import jax
import jax.numpy as jnp
from jax.experimental import pallas as pl
from jax.experimental.pallas import tpu as pltpu


def _mmoe_body(x_ref, we_ref, be_ref, wg_ref, wt_ref, bt_ref, out_ref, gates_ref):
    E = gates_ref.shape[0]
    T = gates_ref.shape[2]
    e = pl.program_id(0)
    j = pl.program_id(1)
    nj = pl.num_programs(1)

    @pl.when(jnp.logical_and(e == 0, j == 0))
    def _init():
        gm = jnp.dot(x_ref[...], wg_ref[...])
        for t in range(T):
            lg = gm[:, t * E:(t + 1) * E]
            m = jnp.max(lg, axis=1, keepdims=True)
            ex = jnp.exp(lg - m)
            sm = ex / jnp.sum(ex, axis=1, keepdims=True)
            for e2 in range(E):
                gates_ref[e2, :, t:t + 1] = sm[:, e2:e2 + 1]
        out_ref[...] = jnp.zeros_like(out_ref)

    h = jnp.maximum(jnp.dot(x_ref[...], we_ref[0]) + be_ref[0], 0.0)
    p = jnp.dot(h, wt_ref[...])
    out_ref[...] += gates_ref[e] * p

    @pl.when(jnp.logical_and(e == E - 1, j == nj - 1))
    def _fini():
        out_ref[...] += bt_ref[...]


def kernel(x, We, be, Wg, Wt, bt):
    N, D = x.shape
    E, _, F = We.shape
    T = Wg.shape[0]
    bf = min(1024, F)
    J = F // bf

    wg_flat = jnp.transpose(Wg, (1, 0, 2)).reshape(D, T * E)
    wt_flat = jnp.transpose(Wt[:, :, 0], (1, 0))
    be3 = be.reshape(E, 1, F)
    bt_row = bt.reshape(1, T)

    out = pl.pallas_call(
        _mmoe_body,
        grid=(E, J),
        in_specs=[
            pl.BlockSpec((N, D), lambda e, j: (0, 0)),
            pl.BlockSpec((1, D, bf), lambda e, j: (e, 0, j)),
            pl.BlockSpec((1, 1, bf), lambda e, j: (e, 0, j)),
            pl.BlockSpec((D, T * E), lambda e, j: (0, 0)),
            pl.BlockSpec((bf, T), lambda e, j: (j, 0)),
            pl.BlockSpec((1, T), lambda e, j: (0, 0)),
        ],
        out_specs=pl.BlockSpec((N, T), lambda e, j: (0, 0)),
        out_shape=jax.ShapeDtypeStruct((N, T), jnp.float32),
        scratch_shapes=[pltpu.VMEM((E, N, T), jnp.float32)],
        compiler_params=pltpu.CompilerParams(
            dimension_semantics=("arbitrary", "arbitrary"),
        ),
    )(x, We, be3, wg_flat, wt_flat, bt_row)

    return jnp.transpose(out)[:, :, None]
